# baseline (speedup 1.0000x reference)
import jax
import jax.numpy as jnp
from jax import lax
from jax.experimental import pallas as pl
from jax.experimental.pallas import tpu as pltpu
from jax.experimental.pallas import tpu_sc as plsc

BATCH = 256
D = 128
OD = 2 * D
L = 16
NV = D // L
NC, NS = 2, 16
NW = NC * NS
SPW = BATCH // NW
CHUNK = 448
BSTEPS = 9


def _feat_copy(feat_hbm, buf_v, sems, b, parity):
    return pltpu.make_async_copy(
        feat_hbm.at[pl.ds(b * D, CHUNK * D)],
        buf_v.at[pl.ds(parity * (CHUNK * D), CHUNK * D)],
        sems.at[0, parity])


def _memb_copy(memb_hbm, memb_v, sems, b, parity):
    return pltpu.make_async_copy(
        memb_hbm.at[pl.ds(b, CHUNK)],
        memb_v.at[pl.ds(parity * (CHUNK + L), CHUNK)],
        sems.at[1, parity])


def _body(feat_hbm, memb_hbm, bounds_hbm, out_hbm, bounds_v, memb_v, buf_v,
          out_v, sems):
    n_rows = memb_hbm.shape[0]
    c = lax.axis_index("c")
    s = lax.axis_index("s")
    w = s * NC + c

    pltpu.sync_copy(bounds_hbm, bounds_v.at[pl.ds(0, BATCH // SPW + 1)])
    bvec = bounds_v[pl.ds(w, L)]
    start = bvec[0]
    end = bvec[1]
    seg_base = w * SPW

    ninf_vec = jnp.full((L,), -jnp.inf, jnp.float32)
    zeros = jnp.zeros((L,), jnp.float32)
    for l in range(SPW):
        for j in range(NV):
            out_v[l, pl.ds(L * j, L)] = zeros
            out_v[l, pl.ds(D + L * j, L)] = ninf_vec

    astart = (start // 8) * 8
    nchunks = (end - astart + CHUNK - 1) // CHUNK

    def dma_base(ci):
        return jnp.minimum(astart + ci * CHUNK, n_rows - CHUNK)

    @pl.when(nchunks > 0)
    def _():
        _feat_copy(feat_hbm, buf_v, sems, dma_base(0), 0).start()
        _memb_copy(memb_hbm, memb_v, sems, dma_base(0), 0).start()

    def chunk_body(ci, cnts):
        parity = lax.rem(ci, 2)
        w0 = astart + ci * CHUNK
        b = dma_base(ci)
        _feat_copy(feat_hbm, buf_v, sems, b, parity).wait()
        _memb_copy(memb_hbm, memb_v, sems, b, parity).wait()

        @pl.when(ci + 1 < nchunks)
        def _():
            nparity = lax.rem(ci + 1, 2)
            nb = dma_base(ci + 1)
            _feat_copy(feat_hbm, buf_v, sems, nb, nparity).start()
            _memb_copy(memb_hbm, memb_v, sems, nb, nparity).start()

        lo = jnp.maximum(start, w0) - b
        hi = jnp.minimum(end, w0 + CHUNK) - b
        mo = parity * (CHUNK + L)
        fo = parity * (CHUNK * D)

        first = memb_v[pl.ds(mo + lo, L)][0]
        last = memb_v[pl.ds(mo + jnp.maximum(hi - 1, lo), L)][0]

        new_cnts = []
        p = lo
        for l in range(SPW):
            seg = seg_base + l
            def bsearch(p=p, seg=seg, hi=hi, mo=mo):
                def bstep(_, lh):
                    blo, bhi = lh
                    mid = lax.shift_right_logical(blo + bhi, 1)
                    v = memb_v[pl.ds(mo + mid, L)][0]
                    live = blo < bhi
                    gt = v > seg
                    return (jnp.where(live & ~gt, mid + 1, blo),
                            jnp.where(live & gt, mid, bhi))
                return lax.fori_loop(0, BSTEPS, bstep, (p, hi))[0]

            e = lax.cond(first > seg, lambda p=p: p,
                         lambda seg=seg: lax.cond(
                             last <= seg, lambda hi=hi: hi, bsearch))

            @pl.when(e > p)
            def _(p=p, e=e, l=l):
                regs = [out_v[l, pl.ds(L * j, L)] for j in range(NV)]
                regs += [out_v[l, pl.ds(D + L * j, L)] for j in range(NV)]

                @plsc.parallel_loop(p, e, unroll=8, carry=tuple(regs))
                def row_loop(r, rg):
                    base = fo + r * D
                    row = [buf_v[pl.ds(base + L * j, L)] for j in range(NV)]
                    return tuple(
                        [rg[j] + row[j] for j in range(NV)]
                        + [jnp.maximum(rg[NV + j], row[j])
                           for j in range(NV)])

                for j in range(NV):
                    out_v[l, pl.ds(L * j, L)] = row_loop[j]
                    out_v[l, pl.ds(D + L * j, L)] = row_loop[NV + j]

            new_cnts.append(cnts[l] + (e - p))
            p = e
        return tuple(new_cnts)

    cnts = lax.fori_loop(0, nchunks, chunk_body,
                         tuple(jnp.int32(0) for _ in range(SPW)))

    for l in range(SPW):
        cvec = jnp.full((L,), cnts[l].astype(jnp.float32))
        for j in range(NV):
            out_v[l, pl.ds(L * j, L)] = out_v[l, pl.ds(L * j, L)] / cvec

    pltpu.sync_copy(out_v, out_hbm.at[pl.ds(seg_base, SPW), :])


@jax.jit
def kernel(atom_features, membership):
    n = membership.shape[0]
    edges = jnp.arange(0, BATCH + 1, SPW, dtype=jnp.int32)
    bounds = jnp.sum((membership[None, :] < edges[:, None]).astype(jnp.int32),
                     axis=1, dtype=jnp.int32)

    mesh = plsc.VectorSubcoreMesh(core_axis_name="c", subcore_axis_name="s",
                                  num_cores=NC, num_subcores=NS)
    run = pl.kernel(
        _body,
        out_type=jax.ShapeDtypeStruct((BATCH, OD), jnp.float32),
        mesh=mesh,
        scratch_types=[
            pltpu.VMEM((BATCH // SPW + 16,), jnp.int32),
            pltpu.VMEM((2 * (CHUNK + L),), jnp.int32),
            pltpu.VMEM((2 * CHUNK * D,), jnp.float32),
            pltpu.VMEM((SPW, OD), jnp.float32),
            pltpu.SemaphoreType.DMA((2, 2)),
        ],
    )
    return run(atom_features.reshape(n * D), membership, bounds)


# speedup vs baseline: 1.0231x; 1.0231x over previous
import jax
import jax.numpy as jnp
from jax import lax
from jax.experimental import pallas as pl
from jax.experimental.pallas import tpu as pltpu
from jax.experimental.pallas import tpu_sc as plsc

BATCH = 256
D = 128
OD = 2 * D
L = 16
NV = D // L
NC, NS = 2, 16
NW = NC * NS
SPW = BATCH // NW
CHUNK = 448
HALF = CHUNK // 2
BSTEPS = 9


def _feat_copy(feat_hbm, buf_v, sems, b, parity, rows=CHUNK):
    return pltpu.make_async_copy(
        feat_hbm.at[pl.ds(b * D, rows * D)],
        buf_v.at[pl.ds(parity * (CHUNK * D), rows * D)],
        sems.at[0, parity])


def _memb_copy(memb_hbm, memb_v, sems, b, parity, rows=CHUNK):
    return pltpu.make_async_copy(
        memb_hbm.at[pl.ds(b, rows)],
        memb_v.at[pl.ds(parity * (CHUNK + L), rows)],
        sems.at[1, parity])


def _body(feat_hbm, memb_hbm, bounds_hbm, out_hbm, bounds_v, memb_v, buf_v,
          out_v, sems):
    n_rows = memb_hbm.shape[0]
    c = lax.axis_index("c")
    s = lax.axis_index("s")
    w = s * NC + c

    pltpu.sync_copy(bounds_hbm, bounds_v.at[pl.ds(0, BATCH // SPW + 1)])
    bvec = bounds_v[pl.ds(w, L)]
    start = bvec[0]
    end = bvec[1]
    seg_base = w * SPW

    ninf_vec = jnp.full((L,), -jnp.inf, jnp.float32)
    zeros = jnp.zeros((L,), jnp.float32)
    for l in range(SPW):
        for j in range(NV):
            out_v[l, pl.ds(L * j, L)] = zeros
            out_v[l, pl.ds(D + L * j, L)] = ninf_vec

    astart = (start // 8) * 8
    total = end - astart
    nchunks = jnp.where(
        total > 0,
        1 + (jnp.maximum(total - HALF, 0) + CHUNK - 1) // CHUNK,
        0)

    def win_start(ci):
        return jnp.where(ci == 0, astart, astart + HALF + (ci - 1) * CHUNK)

    def full_base(ci):
        return jnp.minimum(win_start(ci), n_rows - CHUNK)

    @pl.when(nchunks > 0)
    def _():
        b0 = jnp.minimum(astart, n_rows - HALF)
        _feat_copy(feat_hbm, buf_v, sems, b0, 0, HALF).start()
        _memb_copy(memb_hbm, memb_v, sems, b0, 0, HALF).start()

        @pl.when(nchunks > 1)
        def _():
            _feat_copy(feat_hbm, buf_v, sems, full_base(1), 1).start()
            _memb_copy(memb_hbm, memb_v, sems, full_base(1), 1).start()

    def chunk_body(ci, cnts):
        parity = lax.rem(ci, 2)
        w0 = win_start(ci)
        rows_d = jnp.where(ci == 0, HALF, CHUNK)
        b = jnp.minimum(w0, n_rows - rows_d)

        @pl.when(ci == 0)
        def _():
            b0 = jnp.minimum(astart, n_rows - HALF)
            _feat_copy(feat_hbm, buf_v, sems, b0, 0, HALF).wait()
            _memb_copy(memb_hbm, memb_v, sems, b0, 0, HALF).wait()

        @pl.when(ci > 0)
        def _():
            _feat_copy(feat_hbm, buf_v, sems, full_base(ci), parity).wait()
            _memb_copy(memb_hbm, memb_v, sems, full_base(ci), parity).wait()

        lo = jnp.maximum(start, w0) - b
        hi = jnp.minimum(end, w0 + rows_d) - b
        mo = parity * (CHUNK + L)
        fo = parity * (CHUNK * D)

        first = memb_v[pl.ds(mo + lo, L)][0]
        last = memb_v[pl.ds(mo + jnp.maximum(hi - 1, lo), L)][0]

        new_cnts = []
        p = lo
        for l in range(SPW):
            seg = seg_base + l
            def bsearch(p=p, seg=seg, hi=hi, mo=mo):
                def bstep(_, lh):
                    blo, bhi = lh
                    mid = lax.shift_right_logical(blo + bhi, 1)
                    v = memb_v[pl.ds(mo + mid, L)][0]
                    live = blo < bhi
                    gt = v > seg
                    return (jnp.where(live & ~gt, mid + 1, blo),
                            jnp.where(live & gt, mid, bhi))
                return lax.fori_loop(0, BSTEPS, bstep, (p, hi))[0]

            e = lax.cond(first > seg, lambda p=p: p,
                         lambda seg=seg: lax.cond(
                             last <= seg, lambda hi=hi: hi, bsearch))

            @pl.when(e > p)
            def _(p=p, e=e, l=l):
                regs = [out_v[l, pl.ds(L * j, L)] for j in range(NV)]
                regs += [out_v[l, pl.ds(D + L * j, L)] for j in range(NV)]

                @plsc.parallel_loop(p, e, unroll=8, carry=tuple(regs))
                def row_loop(r, rg):
                    base = fo + r * D
                    row = [buf_v[pl.ds(base + L * j, L)] for j in range(NV)]
                    return tuple(
                        [rg[j] + row[j] for j in range(NV)]
                        + [jnp.maximum(rg[NV + j], row[j])
                           for j in range(NV)])

                for j in range(NV):
                    out_v[l, pl.ds(L * j, L)] = row_loop[j]
                    out_v[l, pl.ds(D + L * j, L)] = row_loop[NV + j]

            new_cnts.append(cnts[l] + (e - p))
            p = e

        @pl.when(ci + 2 < nchunks)
        def _():
            _feat_copy(feat_hbm, buf_v, sems, full_base(ci + 2), parity).start()
            _memb_copy(memb_hbm, memb_v, sems, full_base(ci + 2), parity).start()

        return tuple(new_cnts)

    cnts = lax.fori_loop(0, nchunks, chunk_body,
                         tuple(jnp.int32(0) for _ in range(SPW)))

    for l in range(SPW):
        cvec = jnp.full((L,), cnts[l].astype(jnp.float32))
        for j in range(NV):
            out_v[l, pl.ds(L * j, L)] = out_v[l, pl.ds(L * j, L)] / cvec

    pltpu.sync_copy(out_v, out_hbm.at[pl.ds(seg_base, SPW), :])


@jax.jit
def kernel(atom_features, membership):
    n = membership.shape[0]
    edges = jnp.arange(0, BATCH + 1, SPW, dtype=jnp.int32)
    bounds = jnp.sum((membership[None, :] < edges[:, None]).astype(jnp.int32),
                     axis=1, dtype=jnp.int32)

    mesh = plsc.VectorSubcoreMesh(core_axis_name="c", subcore_axis_name="s",
                                  num_cores=NC, num_subcores=NS)
    run = pl.kernel(
        _body,
        out_type=jax.ShapeDtypeStruct((BATCH, OD), jnp.float32),
        mesh=mesh,
        scratch_types=[
            pltpu.VMEM((BATCH // SPW + 16,), jnp.int32),
            pltpu.VMEM((2 * (CHUNK + L),), jnp.int32),
            pltpu.VMEM((2 * CHUNK * D,), jnp.float32),
            pltpu.VMEM((SPW, OD), jnp.float32),
            pltpu.SemaphoreType.DMA((2, 2)),
        ],
    )
    return run(atom_features.reshape(n * D), membership, bounds)


# speedup vs baseline: 1.0339x; 1.0105x over previous
import jax
import jax.numpy as jnp
from jax import lax
from jax.experimental import pallas as pl
from jax.experimental.pallas import tpu as pltpu
from jax.experimental.pallas import tpu_sc as plsc

BATCH = 256
D = 128
OD = 2 * D
L = 16
NV = D // L
NC, NS = 2, 16
NW = NC * NS
SPW = BATCH // NW
CHUNK = 448
HALF = CHUNK // 2
BSTEPS = 9


def _feat_copy(feat_hbm, buf_v, sems, b, parity, rows=CHUNK):
    return pltpu.make_async_copy(
        feat_hbm.at[pl.ds(b * D, rows * D)],
        buf_v.at[pl.ds(parity * (CHUNK * D), rows * D)],
        sems.at[0, parity])


def _memb_copy(memb_hbm, memb_v, sems, b, parity, rows=CHUNK):
    return pltpu.make_async_copy(
        memb_hbm.at[pl.ds(b, rows)],
        memb_v.at[pl.ds(parity * (CHUNK + L), rows)],
        sems.at[1, parity])


def _body(feat_hbm, memb_hbm, bounds_hbm, out_hbm, bounds_v, memb_v, buf_v,
          out_v, sems):
    n_rows = memb_hbm.shape[0]
    c = lax.axis_index("c")
    s = lax.axis_index("s")
    w = s * NC + c

    pltpu.sync_copy(bounds_hbm, bounds_v.at[pl.ds(0, BATCH // SPW + 1)])
    bvec = bounds_v[pl.ds(w, L)]
    start = bvec[0]
    end = bvec[1]
    seg_base = w * SPW

    ninf_vec = jnp.full((L,), -jnp.inf, jnp.float32)
    zeros = jnp.zeros((L,), jnp.float32)
    for l in range(SPW):
        for j in range(NV):
            out_v[l, pl.ds(L * j, L)] = zeros
            out_v[l, pl.ds(D + L * j, L)] = ninf_vec

    astart = (start // 8) * 8
    total = end - astart
    nchunks = jnp.where(
        total > 0,
        1 + (jnp.maximum(total - HALF, 0) + CHUNK - 1) // CHUNK,
        0)

    def win_start(ci):
        return jnp.where(ci == 0, astart, astart + HALF + (ci - 1) * CHUNK)

    def full_base(ci):
        return jnp.minimum(win_start(ci), n_rows - CHUNK)

    @pl.when(nchunks > 0)
    def _():
        b0 = jnp.minimum(astart, n_rows - HALF)
        _feat_copy(feat_hbm, buf_v, sems, b0, 0, HALF).start()
        _memb_copy(memb_hbm, memb_v, sems, b0, 0, HALF).start()

        @pl.when(nchunks > 1)
        def _():
            _feat_copy(feat_hbm, buf_v, sems, full_base(1), 1).start()
            _memb_copy(memb_hbm, memb_v, sems, full_base(1), 1).start()

    def chunk_body(ci, cnts):
        parity = lax.rem(ci, 2)
        w0 = win_start(ci)
        rows_d = jnp.where(ci == 0, HALF, CHUNK)
        b = jnp.minimum(w0, n_rows - rows_d)

        @pl.when(ci == 0)
        def _():
            b0 = jnp.minimum(astart, n_rows - HALF)
            _feat_copy(feat_hbm, buf_v, sems, b0, 0, HALF).wait()
            _memb_copy(memb_hbm, memb_v, sems, b0, 0, HALF).wait()

        @pl.when(ci > 0)
        def _():
            _feat_copy(feat_hbm, buf_v, sems, full_base(ci), parity).wait()
            _memb_copy(memb_hbm, memb_v, sems, full_base(ci), parity).wait()

        lo = jnp.maximum(start, w0) - b
        hi = jnp.minimum(end, w0 + rows_d) - b
        mo = parity * (CHUNK + L)
        fo = parity * (CHUNK * D)

        first = memb_v[pl.ds(mo + lo, L)][0]
        last = memb_v[pl.ds(mo + jnp.maximum(hi - 1, lo), L)][0]

        new_cnts = []
        p = lo
        for l in range(SPW):
            seg = seg_base + l
            def bsearch(p=p, seg=seg, hi=hi, mo=mo):
                def bstep(_, lh):
                    blo, bhi = lh
                    mid = lax.shift_right_logical(blo + bhi, 1)
                    v = memb_v[pl.ds(mo + mid, L)][0]
                    live = blo < bhi
                    gt = v > seg
                    return (jnp.where(live & ~gt, mid + 1, blo),
                            jnp.where(live & gt, mid, bhi))
                return lax.fori_loop(0, BSTEPS, bstep, (p, hi))[0]

            e = lax.cond(first > seg, lambda p=p: p,
                         lambda seg=seg: lax.cond(
                             last <= seg, lambda hi=hi: hi, bsearch))

            @pl.when(e > p)
            def _(p=p, e=e, l=l):
                regs = [out_v[l, pl.ds(L * j, L)] for j in range(NV)]
                regs += [out_v[l, pl.ds(D + L * j, L)] for j in range(NV)]

                @plsc.parallel_loop(p, e, unroll=8, carry=tuple(regs))
                def row_loop(r, rg):
                    base = fo + r * D
                    row = [buf_v[pl.ds(base + L * j, L)] for j in range(NV)]
                    return tuple(
                        [rg[j] + row[j] for j in range(NV)]
                        + [jnp.maximum(rg[NV + j], row[j])
                           for j in range(NV)])

                for j in range(NV):
                    out_v[l, pl.ds(L * j, L)] = row_loop[j]
                    out_v[l, pl.ds(D + L * j, L)] = row_loop[NV + j]

            new_cnts.append(cnts[l] + (e - p))
            p = e

        @pl.when(ci + 2 < nchunks)
        def _():
            _feat_copy(feat_hbm, buf_v, sems, full_base(ci + 2), parity).start()
            _memb_copy(memb_hbm, memb_v, sems, full_base(ci + 2), parity).start()

        return tuple(new_cnts)

    cnts = lax.fori_loop(0, nchunks, chunk_body,
                         tuple(jnp.int32(0) for _ in range(SPW)))

    for l in range(SPW):
        cvec = jnp.full((L,), cnts[l].astype(jnp.float32))
        for j in range(NV):
            out_v[l, pl.ds(L * j, L)] = out_v[l, pl.ds(L * j, L)] / cvec

    pltpu.sync_copy(out_v, out_hbm.at[pl.ds(seg_base, SPW), :])


@jax.jit
def kernel(atom_features, membership):
    n = membership.shape[0]
    edges = jnp.arange(0, BATCH + 1, SPW, dtype=jnp.int32)
    cmp = (membership[None, :] < edges[:, None]).astype(jnp.bfloat16)
    ones = jnp.ones((n,), jnp.bfloat16)
    bounds = lax.dot_general(cmp, ones, (((1,), (0,)), ((), ())),
                             preferred_element_type=jnp.float32
                             ).astype(jnp.int32)

    mesh = plsc.VectorSubcoreMesh(core_axis_name="c", subcore_axis_name="s",
                                  num_cores=NC, num_subcores=NS)
    run = pl.kernel(
        _body,
        out_type=jax.ShapeDtypeStruct((BATCH, OD), jnp.float32),
        mesh=mesh,
        scratch_types=[
            pltpu.VMEM((BATCH // SPW + 16,), jnp.int32),
            pltpu.VMEM((2 * (CHUNK + L),), jnp.int32),
            pltpu.VMEM((2 * CHUNK * D,), jnp.float32),
            pltpu.VMEM((SPW, OD), jnp.float32),
            pltpu.SemaphoreType.DMA((2, 2)),
        ],
    )
    return run(atom_features.reshape(n * D), membership, bounds)


# speedup vs baseline: 1.0359x; 1.0020x over previous
import jax
import jax.numpy as jnp
from jax import lax
from jax.experimental import pallas as pl
from jax.experimental.pallas import tpu as pltpu
from jax.experimental.pallas import tpu_sc as plsc

BATCH = 256
D = 128
OD = 2 * D
L = 16
NV = D // L
NC, NS = 2, 16
NW = NC * NS
SPW = BATCH // NW
CHUNK = 480
HALF = CHUNK // 2
BSTEPS = 9


def _feat_copy(feat_hbm, buf_v, sems, b, parity, rows=CHUNK):
    return pltpu.make_async_copy(
        feat_hbm.at[pl.ds(b * D, rows * D)],
        buf_v.at[pl.ds(parity * (CHUNK * D), rows * D)],
        sems.at[0, parity])


def _memb_copy(memb_hbm, memb_v, sems, b, parity, rows=CHUNK):
    return pltpu.make_async_copy(
        memb_hbm.at[pl.ds(b, rows)],
        memb_v.at[pl.ds(parity * (CHUNK + L), rows)],
        sems.at[1, parity])


def _body(feat_hbm, memb_hbm, bounds_hbm, out_hbm, bounds_v, memb_v, buf_v,
          out_v, sems):
    n_rows = memb_hbm.shape[0]
    c = lax.axis_index("c")
    s = lax.axis_index("s")
    w = s * NC + c

    pltpu.sync_copy(bounds_hbm, bounds_v.at[pl.ds(0, BATCH // SPW + 1)])
    bvec = bounds_v[pl.ds(w, L)]
    start = bvec[0]
    end = bvec[1]
    seg_base = w * SPW

    ninf_vec = jnp.full((L,), -jnp.inf, jnp.float32)
    zeros = jnp.zeros((L,), jnp.float32)
    for l in range(SPW):
        for j in range(NV):
            out_v[l, pl.ds(L * j, L)] = zeros
            out_v[l, pl.ds(D + L * j, L)] = ninf_vec

    astart = (start // 8) * 8
    total = end - astart
    nchunks = jnp.where(
        total > 0,
        1 + (jnp.maximum(total - HALF, 0) + CHUNK - 1) // CHUNK,
        0)

    def win_start(ci):
        return jnp.where(ci == 0, astart, astart + HALF + (ci - 1) * CHUNK)

    def full_base(ci):
        return jnp.minimum(win_start(ci), n_rows - CHUNK)

    @pl.when(nchunks > 0)
    def _():
        b0 = jnp.minimum(astart, n_rows - HALF)
        _feat_copy(feat_hbm, buf_v, sems, b0, 0, HALF).start()
        _memb_copy(memb_hbm, memb_v, sems, b0, 0, HALF).start()

        @pl.when(nchunks > 1)
        def _():
            _feat_copy(feat_hbm, buf_v, sems, full_base(1), 1).start()
            _memb_copy(memb_hbm, memb_v, sems, full_base(1), 1).start()

    def chunk_body(ci, cnts):
        parity = lax.rem(ci, 2)
        w0 = win_start(ci)
        rows_d = jnp.where(ci == 0, HALF, CHUNK)
        b = jnp.minimum(w0, n_rows - rows_d)

        @pl.when(ci == 0)
        def _():
            b0 = jnp.minimum(astart, n_rows - HALF)
            _feat_copy(feat_hbm, buf_v, sems, b0, 0, HALF).wait()
            _memb_copy(memb_hbm, memb_v, sems, b0, 0, HALF).wait()

        @pl.when(ci > 0)
        def _():
            _feat_copy(feat_hbm, buf_v, sems, full_base(ci), parity).wait()
            _memb_copy(memb_hbm, memb_v, sems, full_base(ci), parity).wait()

        lo = jnp.maximum(start, w0) - b
        hi = jnp.minimum(end, w0 + rows_d) - b
        mo = parity * (CHUNK + L)
        fo = parity * (CHUNK * D)

        first = memb_v[pl.ds(mo + lo, L)][0]
        last = memb_v[pl.ds(mo + jnp.maximum(hi - 1, lo), L)][0]

        new_cnts = []
        p = lo
        for l in range(SPW):
            seg = seg_base + l
            def bsearch(p=p, seg=seg, hi=hi, mo=mo):
                def bstep(_, lh):
                    blo, bhi = lh
                    mid = lax.shift_right_logical(blo + bhi, 1)
                    v = memb_v[pl.ds(mo + mid, L)][0]
                    live = blo < bhi
                    gt = v > seg
                    return (jnp.where(live & ~gt, mid + 1, blo),
                            jnp.where(live & gt, mid, bhi))
                return lax.fori_loop(0, BSTEPS, bstep, (p, hi))[0]

            e = lax.cond(first > seg, lambda p=p: p,
                         lambda seg=seg: lax.cond(
                             last <= seg, lambda hi=hi: hi, bsearch))

            @pl.when(e > p)
            def _(p=p, e=e, l=l):
                regs = [out_v[l, pl.ds(L * j, L)] for j in range(NV)]
                regs += [out_v[l, pl.ds(D + L * j, L)] for j in range(NV)]

                @plsc.parallel_loop(p, e, unroll=8, carry=tuple(regs))
                def row_loop(r, rg):
                    base = fo + r * D
                    row = [buf_v[pl.ds(base + L * j, L)] for j in range(NV)]
                    return tuple(
                        [rg[j] + row[j] for j in range(NV)]
                        + [jnp.maximum(rg[NV + j], row[j])
                           for j in range(NV)])

                for j in range(NV):
                    out_v[l, pl.ds(L * j, L)] = row_loop[j]
                    out_v[l, pl.ds(D + L * j, L)] = row_loop[NV + j]

            new_cnts.append(cnts[l] + (e - p))
            p = e

        @pl.when(ci + 2 < nchunks)
        def _():
            _feat_copy(feat_hbm, buf_v, sems, full_base(ci + 2), parity).start()
            _memb_copy(memb_hbm, memb_v, sems, full_base(ci + 2), parity).start()

        return tuple(new_cnts)

    cnts = lax.fori_loop(0, nchunks, chunk_body,
                         tuple(jnp.int32(0) for _ in range(SPW)))

    for l in range(SPW):
        cvec = jnp.full((L,), cnts[l].astype(jnp.float32))
        for j in range(NV):
            out_v[l, pl.ds(L * j, L)] = out_v[l, pl.ds(L * j, L)] / cvec

    pltpu.sync_copy(out_v, out_hbm.at[pl.ds(seg_base, SPW), :])


@jax.jit
def kernel(atom_features, membership):
    n = membership.shape[0]
    edges = jnp.arange(0, BATCH + 1, SPW, dtype=jnp.int32)
    cmp = (membership[None, :] < edges[:, None]).astype(jnp.bfloat16)
    ones = jnp.ones((n,), jnp.bfloat16)
    bounds = lax.dot_general(cmp, ones, (((1,), (0,)), ((), ())),
                             preferred_element_type=jnp.float32
                             ).astype(jnp.int32)

    mesh = plsc.VectorSubcoreMesh(core_axis_name="c", subcore_axis_name="s",
                                  num_cores=NC, num_subcores=NS)
    run = pl.kernel(
        _body,
        out_type=jax.ShapeDtypeStruct((BATCH, OD), jnp.float32),
        mesh=mesh,
        scratch_types=[
            pltpu.VMEM((BATCH // SPW + 16,), jnp.int32),
            pltpu.VMEM((2 * (CHUNK + L),), jnp.int32),
            pltpu.VMEM((2 * CHUNK * D,), jnp.float32),
            pltpu.VMEM((SPW, OD), jnp.float32),
            pltpu.SemaphoreType.DMA((2, 2)),
        ],
    )
    return run(atom_features.reshape(n * D), membership, bounds)
